# baseline (speedup 1.0000x reference)
import functools

import jax
import jax.numpy as jnp
from jax import lax
from jax.experimental import pallas as pl
from jax.experimental.pallas import tpu as pltpu
from jax.experimental.pallas import tpu_sc as plsc

D = 768
C = 16
NSLOT = 4
EPS = 1e-12


@functools.partial(jax.jit, static_argnames=("n_tokens",))
def _gather_words(ids, word_emb, *, n_tokens):
    info = plsc.get_sparse_core_info()
    nw = info.num_cores * info.num_subcores
    n_per_w = n_tokens // nw
    n_chunks = n_per_w // C
    mesh = plsc.VectorSubcoreMesh(core_axis_name="c", subcore_axis_name="s")

    scratch = (
        [pltpu.VMEM((C, D), jnp.float32) for _ in range(NSLOT)]
        + [pltpu.VMEM((C,), jnp.int32) for _ in range(NSLOT)]
        + [pltpu.SemaphoreType.DMA for _ in range(3 * NSLOT)]
    )

    @functools.partial(
        pl.kernel,
        out_type=jax.ShapeDtypeStruct((n_tokens, D), jnp.float32),
        mesh=mesh,
        scratch_types=scratch,
    )
    def k(ids_hbm, word_hbm, out_hbm, *sc):
        rows = sc[0:4]
        idsv = sc[4:8]
        sem_ids = sc[8:12]
        sem_w = sc[12:16]
        sem_o = sc[16:20]

        cid = lax.axis_index("c")
        sid = lax.axis_index("s")
        wid = sid * info.num_cores + cid

        def base_of(kk):
            return wid * n_per_w + kk * C

        def issue_ids(kk, slot):
            pltpu.async_copy(ids_hbm.at[pl.ds(base_of(kk), C)], idsv[slot],
                             sem_ids[slot])

        def wait_ids(slot):
            pltpu.make_async_copy(ids_hbm.at[pl.ds(0, C)], idsv[slot],
                                  sem_ids[slot]).wait()

        def issue_word(slot):
            pltpu.async_copy(word_hbm.at[idsv[slot]], rows[slot],
                             sem_w[slot])

        def wait_word(slot):
            pltpu.make_async_copy(word_hbm.at[idsv[slot]], rows[slot],
                                  sem_w[slot]).wait()

        def issue_out(kk, slot):
            pltpu.async_copy(rows[slot], out_hbm.at[pl.ds(base_of(kk), C)],
                             sem_o[slot])

        def wait_out(slot):
            pltpu.make_async_copy(rows[slot], out_hbm.at[pl.ds(0, C)],
                                  sem_o[slot]).wait()

        issue_ids(0, 0)
        issue_ids(1, 1)
        issue_ids(2, 2)
        wait_ids(0)
        issue_word(0)

        def body(k0, carry):
            for p in range(NSLOT):
                kk = k0 * NSLOT + p
                p3 = (p + 3) % NSLOT
                sl1 = (p + 1) % NSLOT
                sl2 = (p + 2) % NSLOT

                @pl.when(kk + 3 < n_chunks)
                def _(p3=p3, kk=kk):
                    issue_ids(kk + 3, p3)

                @pl.when(kk >= 2)
                def _(sl2=sl2):
                    wait_out(sl2)

                @pl.when(kk + 1 < n_chunks)
                def _(sl1=sl1):
                    wait_ids(sl1)
                    issue_word(sl1)

                wait_word(p)
                issue_out(kk, p)
            return carry

        lax.fori_loop(0, n_chunks // NSLOT, body, 0)
        wait_out((n_chunks - 2) % NSLOT)
        wait_out((n_chunks - 1) % NSLOT)

    return k(ids, word_emb)


def _ln_math(stage_ref, tt_ref, pos_ref, type_ref, gamma_ref, beta_ref,
             out_ref):
    e = stage_ref[0] + pos_ref[...]
    tt = tt_ref[0, 0]
    types = jnp.where(tt[:, None] == 0, type_ref[0][None, :],
                      type_ref[1][None, :])
    e = e + types
    mean = jnp.mean(e, axis=-1, keepdims=True)
    var = jnp.mean(e * e, axis=-1, keepdims=True) - mean * mean
    inv = lax.rsqrt(var + EPS)
    out_ref[0] = (e - mean) * inv * gamma_ref[...] + beta_ref[...]


def _ln_body_first(stage_ref, tt_ref, pos_ref, type_ref, gamma_ref, beta_ref,
                   out_ref):
    _ln_math(stage_ref, tt_ref, pos_ref, type_ref, gamma_ref, beta_ref,
             out_ref)


def _ln_body_second(stage_ref, tt_ref, pos_ref, type_ref, gamma_ref,
                    beta_ref, prev_ref, out_ref):
    del prev_ref
    _ln_math(stage_ref, tt_ref, pos_ref, type_ref, gamma_ref, beta_ref,
             out_ref)


@functools.partial(jax.jit, static_argnames=("total_b", "off"))
def _ln_part(stage, tt3, pos_emb, type_emb, gamma, beta, prev, *,
             total_b, off):
    b, s, _ = stage.shape
    in_specs = [
        pl.BlockSpec((1, s, D), lambda i: (i, 0, 0)),
        pl.BlockSpec((1, 1, s), lambda i: (i, 0, 0)),
        pl.BlockSpec((s, D), lambda i: (0, 0)),
        pl.BlockSpec(type_emb.shape, lambda i: (0, 0)),
        pl.BlockSpec((D,), lambda i: (0,)),
        pl.BlockSpec((D,), lambda i: (0,)),
    ]
    args = [stage, tt3, pos_emb, type_emb, gamma, beta]
    kwargs = {}
    if prev is None:
        body = _ln_body_first
    else:
        body = _ln_body_second
        in_specs.append(pl.BlockSpec(memory_space=pltpu.MemorySpace.HBM))
        args.append(prev)
        kwargs["input_output_aliases"] = {6: 0}
    return pl.pallas_call(
        body,
        grid=(b,),
        in_specs=in_specs,
        out_specs=pl.BlockSpec((1, s, D), lambda i: (i + off, 0, 0)),
        out_shape=jax.ShapeDtypeStruct((total_b, s, D), jnp.float32),
        **kwargs,
    )(*args)


def kernel(input_ids, token_type_ids, attention_mask, word_emb, pos_emb,
           type_emb, gamma, beta):
    b, s = input_ids.shape
    nsplit = 8
    bh = b // nsplit
    ids_f = input_ids.reshape(-1)
    tt3 = token_type_ids.reshape(b, 1, s)
    stages = [
        _gather_words(ids_f[q * bh * s:(q + 1) * bh * s], word_emb,
                      n_tokens=bh * s)
        for q in range(nsplit)
    ]
    out = None
    for q in range(nsplit):
        out = _ln_part(stages[q].reshape(bh, s, D),
                       tt3[q * bh:(q + 1) * bh], pos_emb, type_emb,
                       gamma, beta, out, total_b=b, off=q * bh)
    return out, attention_mask
